# baseline (speedup 1.0000x reference)
import functools

import jax
import jax.numpy as jnp
from jax import lax

_GDN = lax.GatherDimensionNumbers(
    offset_dims=(), collapsed_slice_dims=(0,), start_index_map=(0,))


def _splat_lane(vec, l):
    idx = jnp.full((16, 1), l, jnp.int32)
    return lax.gather(vec, idx, _GDN, slice_sizes=(1,),
                      mode=lax.GatherScatterMode.PROMISE_IN_BOUNDS)
from jax.experimental import pallas as pl
from jax.experimental.pallas import tpu as pltpu
from jax.experimental.pallas import tpu_sc as plsc

N = 10000
F = 128
HID = 512
NC = 2
NS = 16
NW = NC * NS
DCH = 128
SCH = 64
NB = 4
WB = 64
N_PAD = 10240
DPT = N_PAD // NS
RB = 1000


def _make_deg(e_pads):
    kchs = [e // (NW * DCH) for e in e_pads]
    epts = [e // NW for e in e_pads]
    mesh = plsc.VectorSubcoreMesh(core_axis_name="c", subcore_axis_name="s")

    scratch = [pltpu.VMEM_SHARED((N_PAD,), jnp.float32) for _ in range(3)]
    scratch += [pltpu.VMEM((kchs[i], DCH), jnp.int32) for i in range(3)]
    scratch += [pltpu.VMEM((epts[i],), jnp.float32) for i in range(3)]
    scratch += [pltpu.VMEM((DPT,), jnp.float32)]
    scratch += [pltpu.SemaphoreType.DMA] * NB

    @functools.partial(
        pl.kernel,
        out_type=[jax.ShapeDtypeStruct((NC, N_PAD), jnp.float32)] * 3,
        mesh=mesh,
        scratch_types=scratch,
    )
    def deg_kernel(col0, col1, col2, ew0, ew1, ew2, zeros_hbm,
                   out0, out1, out2,
                   acc0, acc1, acc2, ca0, ca1, ca2, ea0, ea1, ea2,
                   bounce, s0, s1, s2, s3):
        cols = [col0, col1, col2]
        ews = [ew0, ew1, ew2]
        outs = [out0, out1, out2]
        accs = [acc0, acc1, acc2]
        cas = [ca0, ca1, ca2]
        eas = [ea0, ea1, ea2]
        ssem = [s0, s1, s2, s3]
        cid = lax.axis_index("c")
        sid = lax.axis_index("s")
        tid = cid * NS + sid
        pltpu.sync_copy(zeros_hbm, bounce)
        for i in range(3):
            pltpu.sync_copy(cols[i].at[tid], cas[i])
            pltpu.sync_copy(ews[i].at[pl.ds(tid * epts[i], epts[i])], eas[i])
            pltpu.sync_copy(bounce, accs[i].at[pl.ds(sid * DPT, DPT)])
        plsc.subcore_barrier()

        for i in range(3):
            kch = kchs[i]

            def sc_desc(k, b, _i=i):
                return pltpu.make_async_copy(
                    eas[_i].at[pl.ds(k * DCH, DCH)],
                    accs[_i].at[cas[_i].at[k]], ssem[b])

            def body(kk, carry, _i=i, _kch=kch):
                for b in range(NB):
                    k = NB * kk + b

                    @pl.when(kk > 0)
                    def _():
                        sc_desc(k - NB, b, _i).wait()

                    sc_desc(k, b, _i).start(add=True)
                return carry

            lax.fori_loop(0, kch // NB, body, 0)
            for b in range(NB):
                sc_desc(kch - NB + b, b, i).wait()
        plsc.subcore_barrier()
        for i in range(3):
            pltpu.sync_copy(accs[i].at[pl.ds(sid * DPT, DPT)], bounce)
            pltpu.sync_copy(bounce, outs[i].at[cid, pl.ds(sid * DPT, DPT)])

    return deg_kernel


def _make_spmm(e_pad, scale):
    ept = e_pad // NW
    kch = ept // SCH
    mesh = plsc.VectorSubcoreMesh(core_axis_name="c", subcore_axis_name="s")

    scratch = [pltpu.VMEM_SHARED((N_PAD, F), jnp.float32)]
    scratch += [pltpu.VMEM((SCH,), jnp.int32) for _ in range(NB)]
    scratch += [pltpu.VMEM((SCH,), jnp.int32) for _ in range(2 * NB)]
    scratch += [pltpu.VMEM((SCH,), jnp.float32) for _ in range(NB)]
    scratch += [pltpu.VMEM((SCH, F), jnp.float32) for _ in range(NB)]
    scratch += [pltpu.SemaphoreType.DMA] * (6 * NB)

    @functools.partial(
        pl.kernel,
        out_type=jax.ShapeDtypeStruct((NC, N_PAD, F), jnp.float32),
        mesh=mesh,
        scratch_types=scratch,
    )
    def spmm_kernel(y_hbm, row2, col2, ew2, zeros_hbm, out_hbm,
                    acc_sh, *bufs):
        ibufs = list(bufs[0:NB])
        cbufs = list(bufs[NB:3 * NB])
        ebufs = list(bufs[3 * NB:4 * NB])
        rbufs = list(bufs[4 * NB:5 * NB])
        isem = list(bufs[5 * NB:6 * NB])
        csem = list(bufs[6 * NB:8 * NB])
        esem = list(bufs[8 * NB:9 * NB])
        gsem = list(bufs[9 * NB:10 * NB])
        ssem = list(bufs[10 * NB:11 * NB])
        cid = lax.axis_index("c")
        sid = lax.axis_index("s")
        tid = cid * NS + sid
        r0 = sid * DPT
        pltpu.sync_copy(zeros_hbm, rbufs[0])
        for m in range(DPT // WB):
            pltpu.sync_copy(rbufs[0], acc_sh.at[pl.ds(r0 + m * WB, WB)])

        def i_desc(k, b):
            return pltpu.make_async_copy(row2.at[tid * kch + k],
                                         ibufs[b], isem[b])

        def c_desc(k, cb):
            return pltpu.make_async_copy(col2.at[tid * kch + k],
                                         cbufs[cb], csem[cb])

        def e_desc(k, b):
            return pltpu.make_async_copy(ew2.at[tid * kch + k],
                                         ebufs[b], esem[b])

        def g_desc(k, b):
            return pltpu.make_async_copy(
                y_hbm.at[ibufs[b]], rbufs[b], gsem[b])

        def s_desc(k, b, cb):
            return pltpu.make_async_copy(
                rbufs[b], acc_sh.at[cbufs[cb]], ssem[b])

        for k in range(NB):
            i_desc(k, k).start()
            if scale:
                e_desc(k, k).start()
        for k in range(6):
            c_desc(k, k).start()
        for k in range(2):
            i_desc(k, k).wait()
            g_desc(k, k).start()
        plsc.subcore_barrier()

        def body(kk, carry):
            for b in range(2 * NB):
                k = 2 * NB * kk + b
                rb = b % NB
                g_desc(k, rb).wait()
                if scale:
                    e_desc(k, rb).wait()

                    def sc_body(g, c2, _rb=rb):
                        ewv = ebufs[_rb][pl.ds(g * 16, 16)]
                        for l in range(16):
                            w = _splat_lane(ewv, l)
                            e = g * 16 + l
                            for j in range(F // 16):
                                sl = pl.ds(j * 16, 16)
                                rbufs[_rb][e, sl] = rbufs[_rb][e, sl] * w
                        return c2

                    lax.fori_loop(0, SCH // 16, sc_body, 0)
                c_desc(k, b).wait()
                s_desc(k, rb, b).start(add=True)

                @pl.when(k + NB < kch)
                def _():
                    i_desc(k + NB, rb).start()
                    if scale:
                        e_desc(k + NB, rb).start()

                nxt = (b + 2) % NB

                @pl.when(jnp.logical_and(k >= 2, k + 2 < kch))
                def _():
                    s_desc(k - 2, nxt, (b + 6) % (2 * NB)).wait()

                @pl.when(k + 6 < kch)
                def _():
                    c_desc(k + 6, (b + 6) % (2 * NB)).start()

                @pl.when(k + 2 < kch)
                def _():
                    i_desc(k + 2, nxt).wait()
                    g_desc(k + 2, nxt).start()
            return carry

        lax.fori_loop(0, kch // (2 * NB), body, 0)
        for j in range(NB):
            s_desc(kch - NB + j, j, NB + j).wait()
        plsc.subcore_barrier()
        for m in range(DPT // WB):
            pltpu.sync_copy(acc_sh.at[pl.ds(r0 + m * WB, WB)], rbufs[0])
            pltpu.sync_copy(rbufs[0], out_hbm.at[cid, pl.ds(r0 + m * WB, WB)])

    return spmm_kernel


def _dinv_from(degp):
    deg = degp[:, 0] + degp[:, 1] + 1.0
    dinv = jnp.where(deg > 0, lax.rsqrt(jnp.maximum(deg, 1e-12)), 0.0)
    return deg, dinv


def _tc_pre_body(degs_ref, degd_ref, degc_ref, xr_ref, xa_ref,
                 ys_ref, yd_ref, yc_ref):
    _, dinv_s = _dinv_from(degs_ref[...])
    _, dinv_d = _dinv_from(degd_ref[...])
    _, dinv_c = _dinv_from(degc_ref[...])
    ys_ref[...] = xr_ref[...] * dinv_s[:, None]
    yd_ref[...] = xr_ref[...] * dinv_d[:, None]
    yc_ref[...] = xa_ref[...] * dinv_c[:, None]


def _tc_mid_body(p_ref, x_ref, degp_ref, w1_ref, b1_ref, w2_ref,
                 ys_ref, ysc_ref):
    deg, dinv = _dinv_from(degp_ref[...])
    p = p_ref[0] + p_ref[1]
    agg = p * dinv[:, None] + x_ref[...] / deg[:, None]
    xs = jnp.maximum(
        jnp.dot(agg, w1_ref[...], preferred_element_type=jnp.float32)
        + b1_ref[...], 0.0)
    ys = jnp.dot(xs, w2_ref[...], preferred_element_type=jnp.float32)
    ys_ref[...] = ys
    ysc_ref[...] = ys * dinv[:, None]


def _tc_final_body(qs_ref, qd_ref, pc_ref, ys_ref, yd_ref, xa_ref,
                   degs_ref, degd_ref, degc_ref,
                   wp3_ref, bp3_ref, bs_ref, bd_ref,
                   wf1a_ref, wf1b_ref, bf1_ref,
                   wf2a_ref, wf2b_ref, bf2_ref,
                   xsim_ref, xdist_ref, fused_ref, fusedpro_ref, pro_ref):
    deg_s, dinv_s = _dinv_from(degs_ref[...])
    deg_d, dinv_d = _dinv_from(degd_ref[...])
    deg_c, dinv_c = _dinv_from(degc_ref[...])
    qs = qs_ref[0] + qs_ref[1]
    qd = qd_ref[0] + qd_ref[1]
    pc = pc_ref[0] + pc_ref[1]
    x_sim = (qs * dinv_s[:, None]
             + ys_ref[...] / deg_s[:, None] + bs_ref[...])
    x_dist = (qd * dinv_d[:, None]
              + yd_ref[...] / deg_d[:, None] + bd_ref[...])
    aggc = (pc * dinv_c[:, None]
            + xa_ref[...] / deg_c[:, None])
    pro = (jnp.dot(aggc, wp3_ref[...], preferred_element_type=jnp.float32)
           + bp3_ref[...])
    fused = (jnp.dot(x_sim, wf1a_ref[...], preferred_element_type=jnp.float32)
             + jnp.dot(x_dist, wf1b_ref[...],
                       preferred_element_type=jnp.float32)
             + bf1_ref[...])
    fused_pro = (jnp.dot(fused, wf2a_ref[...],
                         preferred_element_type=jnp.float32)
                 + jnp.dot(pro, wf2b_ref[...],
                           preferred_element_type=jnp.float32)
                 + bf2_ref[...])
    xsim_ref[...] = x_sim
    xdist_ref[...] = x_dist
    fused_ref[...] = fused
    fusedpro_ref[...] = fused_pro
    pro_ref[...] = pro


def _row_spec(width):
    return pl.BlockSpec((RB, width), lambda i: (i, 0))


def _part_spec():
    return pl.BlockSpec((2, RB, F), lambda i: (0, i, 0))


def _deg_spec():
    return pl.BlockSpec((RB, 2), lambda i: (i, 0))


def _full_spec(shape):
    nd = len(shape)
    return pl.BlockSpec(shape, lambda i, _n=nd: (0,) * _n)


def _pad_edges(edge_index, ew, e_pad):
    e = ew.shape[0]
    row = edge_index[0]
    col = edge_index[1]
    pad = e_pad - e
    if pad:
        row = jnp.concatenate([row, jnp.zeros((pad,), dtype=row.dtype)])
        col = jnp.concatenate([col, jnp.full((pad,), N, dtype=col.dtype)])
        ew = jnp.concatenate([ew, jnp.zeros((pad,), dtype=ew.dtype)])
    return row, col, ew


def _round_up(e, m):
    return ((e + m - 1) // m) * m


def kernel(x_RNA, x_ADT, sim_edge_index, sim_edge_weight, dist_edge_index,
           dist_edge_weight, common_edge_index, common_edge_weight,
           W_r1, b_r1, W_r2, b_r2, W_p3, b_p3, W_sim, b_sim, W_dist, b_dist,
           W_f1, b_f1, W_f2, b_f2):
    f32 = jnp.float32
    grain = NW * DCH * NB
    es_pad = _round_up(sim_edge_weight.shape[0], grain)
    ed_pad = _round_up(dist_edge_weight.shape[0], grain)
    ec_pad = _round_up(common_edge_weight.shape[0], grain)
    row_s, col_s, ew_s = _pad_edges(sim_edge_index, sim_edge_weight, es_pad)
    row_d, col_d, ew_d = _pad_edges(dist_edge_index, dist_edge_weight, ed_pad)
    row_c, col_c, ew_c = _pad_edges(common_edge_index, common_edge_weight,
                                    ec_pad)

    def _deg_view(col, e_pad):
        return col.reshape(NW, e_pad // (NW * DCH), DCH)

    def _sp_view(a):
        return a.reshape(-1, SCH)

    zeros_deg = jnp.zeros((DPT,), f32)
    zeros_row = jnp.zeros((WB, F), f32)

    deg_s, deg_d, deg_c = _make_deg((es_pad, ed_pad, ec_pad))(
        _deg_view(col_s, es_pad), _deg_view(col_d, ed_pad),
        _deg_view(col_c, ec_pad), ew_s, ew_d, ew_c, zeros_deg)
    degT_s = jnp.swapaxes(deg_s, 0, 1)[:N]
    degT_d = jnp.swapaxes(deg_d, 0, 1)[:N]
    degT_c = jnp.swapaxes(deg_c, 0, 1)[:N]

    grid = (N // RB,)
    y_s, y_d, y_c = pl.pallas_call(
        _tc_pre_body,
        grid=grid,
        in_specs=[_deg_spec(), _deg_spec(), _deg_spec(),
                  _row_spec(F), _row_spec(F)],
        out_specs=[_row_spec(F)] * 3,
        out_shape=[jax.ShapeDtypeStruct((N, F), f32)] * 3,
    )(degT_s, degT_d, degT_c, x_RNA, x_ADT)

    spmm_sim = _make_spmm(es_pad, True)
    spmm_dist = _make_spmm(ed_pad, True)
    spmm_com = _make_spmm(ec_pad, False)

    p_s = spmm_sim(y_s, _sp_view(row_s), _sp_view(col_s),
                   _sp_view(ew_s), zeros_row)
    p_d = spmm_dist(y_d, _sp_view(row_d), _sp_view(col_d),
                    _sp_view(ew_d), zeros_row)
    p_c = spmm_com(y_c, _sp_view(row_c), _sp_view(col_c),
                   _sp_view(ew_c), zeros_row)

    def mid(p, degT, w1, b1, w2):
        return pl.pallas_call(
            _tc_mid_body,
            grid=grid,
            in_specs=[_part_spec(), _row_spec(F), _deg_spec(),
                      _full_spec((F, HID)), _full_spec((1, HID)),
                      _full_spec((HID, F))],
            out_specs=[_row_spec(F), _row_spec(F)],
            out_shape=[jax.ShapeDtypeStruct((N, F), f32)] * 2,
        )(p, x_RNA, degT, w1, b1.reshape(1, HID), w2)

    ys, ysc = mid(p_s, degT_s, W_r1, b_r1, W_sim)
    yd, ydc = mid(p_d, degT_d, W_r2, b_r2, W_dist)

    q_s = spmm_sim(ysc, _sp_view(row_s), _sp_view(col_s),
                   _sp_view(ew_s), zeros_row)
    q_d = spmm_dist(ydc, _sp_view(row_d), _sp_view(col_d),
                    _sp_view(ew_d), zeros_row)

    outs = pl.pallas_call(
        _tc_final_body,
        grid=grid,
        in_specs=[_part_spec(), _part_spec(), _part_spec(),
                  _row_spec(F), _row_spec(F), _row_spec(F),
                  _deg_spec(), _deg_spec(), _deg_spec(),
                  _full_spec((F, F)), _full_spec((1, F)),
                  _full_spec((1, F)), _full_spec((1, F)),
                  _full_spec((F, F)), _full_spec((F, F)), _full_spec((1, F)),
                  _full_spec((F, F)), _full_spec((F, F)), _full_spec((1, F))],
        out_specs=[_row_spec(F)] * 5,
        out_shape=[jax.ShapeDtypeStruct((N, F), f32)] * 5,
    )(q_s, q_d, p_c, ys, yd, x_ADT, degT_s, degT_d, degT_c,
      W_p3, b_p3.reshape(1, F), b_sim.reshape(1, F), b_dist.reshape(1, F),
      W_f1[:F], W_f1[F:], b_f1.reshape(1, F),
      W_f2[:F], W_f2[F:], b_f2.reshape(1, F))
    x_sim, x_dist, fused, fused_pro, pro = outs
    return (x_sim, x_dist, fused, fused_pro, pro)


# speedup vs baseline: 1.0556x; 1.0556x over previous
import functools

import jax
import jax.numpy as jnp
from jax import lax

_GDN = lax.GatherDimensionNumbers(
    offset_dims=(), collapsed_slice_dims=(0,), start_index_map=(0,))


def _splat_lane(vec, l):
    idx = jnp.full((16, 1), l, jnp.int32)
    return lax.gather(vec, idx, _GDN, slice_sizes=(1,),
                      mode=lax.GatherScatterMode.PROMISE_IN_BOUNDS)
from jax.experimental import pallas as pl
from jax.experimental.pallas import tpu as pltpu
from jax.experimental.pallas import tpu_sc as plsc

N = 10000
F = 128
HID = 512
NC = 2
NS = 16
NW = NC * NS
DCH = 128
SCH = 64
NB = 4
WB = 64
N_PAD = 10240
DPT = N_PAD // NS
RB = 1000


def _make_deg(e_pads):
    kchs = [e // (NW * DCH) for e in e_pads]
    epts = [e // NW for e in e_pads]
    mesh = plsc.VectorSubcoreMesh(core_axis_name="c", subcore_axis_name="s")

    scratch = [pltpu.VMEM_SHARED((N_PAD,), jnp.float32) for _ in range(3)]
    scratch += [pltpu.VMEM((kchs[i], DCH), jnp.int32) for i in range(3)]
    scratch += [pltpu.VMEM((epts[i],), jnp.float32) for i in range(3)]
    scratch += [pltpu.VMEM((DPT,), jnp.float32)]
    scratch += [pltpu.SemaphoreType.DMA] * NB

    @functools.partial(
        pl.kernel,
        out_type=[jax.ShapeDtypeStruct((NC, N_PAD), jnp.float32)] * 3,
        mesh=mesh,
        scratch_types=scratch,
    )
    def deg_kernel(col0, col1, col2, ew0, ew1, ew2, zeros_hbm,
                   out0, out1, out2,
                   acc0, acc1, acc2, ca0, ca1, ca2, ea0, ea1, ea2,
                   bounce, s0, s1, s2, s3):
        cols = [col0, col1, col2]
        ews = [ew0, ew1, ew2]
        outs = [out0, out1, out2]
        accs = [acc0, acc1, acc2]
        cas = [ca0, ca1, ca2]
        eas = [ea0, ea1, ea2]
        ssem = [s0, s1, s2, s3]
        cid = lax.axis_index("c")
        sid = lax.axis_index("s")
        tid = cid * NS + sid
        pltpu.sync_copy(zeros_hbm, bounce)
        for i in range(3):
            pltpu.sync_copy(cols[i].at[tid], cas[i])
            pltpu.sync_copy(ews[i].at[pl.ds(tid * epts[i], epts[i])], eas[i])
            pltpu.sync_copy(bounce, accs[i].at[pl.ds(sid * DPT, DPT)])
        plsc.subcore_barrier()

        for i in range(3):
            kch = kchs[i]

            def sc_desc(k, b, _i=i):
                return pltpu.make_async_copy(
                    eas[_i].at[pl.ds(k * DCH, DCH)],
                    accs[_i].at[cas[_i].at[k]], ssem[b])

            def body(kk, carry, _i=i, _kch=kch):
                for b in range(NB):
                    k = NB * kk + b

                    @pl.when(kk > 0)
                    def _():
                        sc_desc(k - NB, b, _i).wait()

                    sc_desc(k, b, _i).start(add=True)
                return carry

            lax.fori_loop(0, kch // NB, body, 0)
            for b in range(NB):
                sc_desc(kch - NB + b, b, i).wait()
        plsc.subcore_barrier()
        for i in range(3):
            pltpu.sync_copy(accs[i].at[pl.ds(sid * DPT, DPT)], bounce)
            pltpu.sync_copy(bounce, outs[i].at[cid, pl.ds(sid * DPT, DPT)])

    return deg_kernel


def _make_spmm(e_pad, scale):
    ept = e_pad // NW
    kch = ept // SCH
    mesh = plsc.VectorSubcoreMesh(core_axis_name="c", subcore_axis_name="s")

    scratch = [pltpu.VMEM_SHARED((N_PAD, F), jnp.float32)]
    scratch += [pltpu.VMEM((SCH,), jnp.int32) for _ in range(NB)]
    scratch += [pltpu.VMEM((SCH,), jnp.int32) for _ in range(2 * NB)]
    scratch += [pltpu.VMEM((SCH,), jnp.float32) for _ in range(NB)]
    scratch += [pltpu.VMEM((SCH, F), jnp.float32) for _ in range(NB)]
    scratch += [pltpu.SemaphoreType.DMA] * (6 * NB)

    @functools.partial(
        pl.kernel,
        out_type=jax.ShapeDtypeStruct((NC, N_PAD, F), jnp.float32),
        mesh=mesh,
        scratch_types=scratch,
    )
    def spmm_kernel(y_hbm, row2, col2, ew2, zeros_hbm, out_hbm,
                    acc_sh, *bufs):
        ibufs = list(bufs[0:NB])
        cbufs = list(bufs[NB:3 * NB])
        ebufs = list(bufs[3 * NB:4 * NB])
        rbufs = list(bufs[4 * NB:5 * NB])
        isem = list(bufs[5 * NB:6 * NB])
        csem = list(bufs[6 * NB:8 * NB])
        esem = list(bufs[8 * NB:9 * NB])
        gsem = list(bufs[9 * NB:10 * NB])
        ssem = list(bufs[10 * NB:11 * NB])
        cid = lax.axis_index("c")
        sid = lax.axis_index("s")
        tid = cid * NS + sid
        r0 = sid * DPT
        pltpu.sync_copy(zeros_hbm, rbufs[0])
        for m in range(DPT // WB):
            pltpu.sync_copy(rbufs[0], acc_sh.at[pl.ds(r0 + m * WB, WB)])

        def i_desc(k, b):
            return pltpu.make_async_copy(row2.at[tid * kch + k],
                                         ibufs[b], isem[b])

        def c_desc(k, cb):
            return pltpu.make_async_copy(col2.at[tid * kch + k],
                                         cbufs[cb], csem[cb])

        def e_desc(k, b):
            return pltpu.make_async_copy(ew2.at[tid * kch + k],
                                         ebufs[b], esem[b])

        def g_desc(k, b):
            return pltpu.make_async_copy(
                y_hbm.at[ibufs[b]], rbufs[b], gsem[b])

        def s_desc(k, b, cb):
            return pltpu.make_async_copy(
                rbufs[b], acc_sh.at[cbufs[cb]], ssem[b])

        for k in range(NB):
            i_desc(k, k).start()
            if scale:
                e_desc(k, k).start()
        for k in range(6):
            c_desc(k, k).start()
        for k in range(2):
            i_desc(k, k).wait()
            g_desc(k, k).start()
        plsc.subcore_barrier()

        def body(kk, carry):
            for b in range(2 * NB):
                k = 2 * NB * kk + b
                rb = b % NB
                g_desc(k, rb).wait()
                if scale:
                    e_desc(k, rb).wait()

                    @functools.partial(
                        plsc.parallel_loop, 0, SCH // 16, unroll=2)
                    def _(g, _rb=rb):
                        ewv = ebufs[_rb][pl.ds(g * 16, 16)]
                        for l in range(16):
                            w = ewv[l]
                            e = g * 16 + l
                            for j in range(F // 16):
                                sl = pl.ds(j * 16, 16)
                                rbufs[_rb][e, sl] = rbufs[_rb][e, sl] * w
                c_desc(k, b).wait()
                s_desc(k, rb, b).start(add=True)

                @pl.when(k + NB < kch)
                def _():
                    i_desc(k + NB, rb).start()
                    if scale:
                        e_desc(k + NB, rb).start()

                nxt = (b + 2) % NB

                @pl.when(jnp.logical_and(k >= 2, k + 2 < kch))
                def _():
                    s_desc(k - 2, nxt, (b + 6) % (2 * NB)).wait()

                @pl.when(k + 6 < kch)
                def _():
                    c_desc(k + 6, (b + 6) % (2 * NB)).start()

                @pl.when(k + 2 < kch)
                def _():
                    i_desc(k + 2, nxt).wait()
                    g_desc(k + 2, nxt).start()
            return carry

        lax.fori_loop(0, kch // (2 * NB), body, 0)
        for j in range(NB):
            s_desc(kch - NB + j, j, NB + j).wait()
        plsc.subcore_barrier()
        for m in range(DPT // WB):
            pltpu.sync_copy(acc_sh.at[pl.ds(r0 + m * WB, WB)], rbufs[0])
            pltpu.sync_copy(rbufs[0], out_hbm.at[cid, pl.ds(r0 + m * WB, WB)])

    return spmm_kernel


def _dinv_from(degp):
    deg = degp[:, 0] + degp[:, 1] + 1.0
    dinv = jnp.where(deg > 0, lax.rsqrt(jnp.maximum(deg, 1e-12)), 0.0)
    return deg, dinv


def _tc_pre_body(degs_ref, degd_ref, degc_ref, xr_ref, xa_ref,
                 ys_ref, yd_ref, yc_ref):
    _, dinv_s = _dinv_from(degs_ref[...])
    _, dinv_d = _dinv_from(degd_ref[...])
    _, dinv_c = _dinv_from(degc_ref[...])
    ys_ref[...] = xr_ref[...] * dinv_s[:, None]
    yd_ref[...] = xr_ref[...] * dinv_d[:, None]
    yc_ref[...] = xa_ref[...] * dinv_c[:, None]


def _tc_mid_body(p_ref, x_ref, degp_ref, w1_ref, b1_ref, w2_ref,
                 ys_ref, ysc_ref):
    deg, dinv = _dinv_from(degp_ref[...])
    p = p_ref[0] + p_ref[1]
    agg = p * dinv[:, None] + x_ref[...] / deg[:, None]
    xs = jnp.maximum(
        jnp.dot(agg, w1_ref[...], preferred_element_type=jnp.float32)
        + b1_ref[...], 0.0)
    ys = jnp.dot(xs, w2_ref[...], preferred_element_type=jnp.float32)
    ys_ref[...] = ys
    ysc_ref[...] = ys * dinv[:, None]


def _tc_final_body(qs_ref, qd_ref, pc_ref, ys_ref, yd_ref, xa_ref,
                   degs_ref, degd_ref, degc_ref,
                   wp3_ref, bp3_ref, bs_ref, bd_ref,
                   wf1a_ref, wf1b_ref, bf1_ref,
                   wf2a_ref, wf2b_ref, bf2_ref,
                   xsim_ref, xdist_ref, fused_ref, fusedpro_ref, pro_ref):
    deg_s, dinv_s = _dinv_from(degs_ref[...])
    deg_d, dinv_d = _dinv_from(degd_ref[...])
    deg_c, dinv_c = _dinv_from(degc_ref[...])
    qs = qs_ref[0] + qs_ref[1]
    qd = qd_ref[0] + qd_ref[1]
    pc = pc_ref[0] + pc_ref[1]
    x_sim = (qs * dinv_s[:, None]
             + ys_ref[...] / deg_s[:, None] + bs_ref[...])
    x_dist = (qd * dinv_d[:, None]
              + yd_ref[...] / deg_d[:, None] + bd_ref[...])
    aggc = (pc * dinv_c[:, None]
            + xa_ref[...] / deg_c[:, None])
    pro = (jnp.dot(aggc, wp3_ref[...], preferred_element_type=jnp.float32)
           + bp3_ref[...])
    fused = (jnp.dot(x_sim, wf1a_ref[...], preferred_element_type=jnp.float32)
             + jnp.dot(x_dist, wf1b_ref[...],
                       preferred_element_type=jnp.float32)
             + bf1_ref[...])
    fused_pro = (jnp.dot(fused, wf2a_ref[...],
                         preferred_element_type=jnp.float32)
                 + jnp.dot(pro, wf2b_ref[...],
                           preferred_element_type=jnp.float32)
                 + bf2_ref[...])
    xsim_ref[...] = x_sim
    xdist_ref[...] = x_dist
    fused_ref[...] = fused
    fusedpro_ref[...] = fused_pro
    pro_ref[...] = pro


def _row_spec(width):
    return pl.BlockSpec((RB, width), lambda i: (i, 0))


def _part_spec():
    return pl.BlockSpec((2, RB, F), lambda i: (0, i, 0))


def _deg_spec():
    return pl.BlockSpec((RB, 2), lambda i: (i, 0))


def _full_spec(shape):
    nd = len(shape)
    return pl.BlockSpec(shape, lambda i, _n=nd: (0,) * _n)


def _pad_edges(edge_index, ew, e_pad):
    e = ew.shape[0]
    row = edge_index[0]
    col = edge_index[1]
    pad = e_pad - e
    if pad:
        row = jnp.concatenate([row, jnp.zeros((pad,), dtype=row.dtype)])
        col = jnp.concatenate([col, jnp.full((pad,), N, dtype=col.dtype)])
        ew = jnp.concatenate([ew, jnp.zeros((pad,), dtype=ew.dtype)])
    return row, col, ew


def _round_up(e, m):
    return ((e + m - 1) // m) * m


def kernel(x_RNA, x_ADT, sim_edge_index, sim_edge_weight, dist_edge_index,
           dist_edge_weight, common_edge_index, common_edge_weight,
           W_r1, b_r1, W_r2, b_r2, W_p3, b_p3, W_sim, b_sim, W_dist, b_dist,
           W_f1, b_f1, W_f2, b_f2):
    f32 = jnp.float32
    grain = NW * DCH * NB
    es_pad = _round_up(sim_edge_weight.shape[0], grain)
    ed_pad = _round_up(dist_edge_weight.shape[0], grain)
    ec_pad = _round_up(common_edge_weight.shape[0], grain)
    row_s, col_s, ew_s = _pad_edges(sim_edge_index, sim_edge_weight, es_pad)
    row_d, col_d, ew_d = _pad_edges(dist_edge_index, dist_edge_weight, ed_pad)
    row_c, col_c, ew_c = _pad_edges(common_edge_index, common_edge_weight,
                                    ec_pad)

    def _deg_view(col, e_pad):
        return col.reshape(NW, e_pad // (NW * DCH), DCH)

    def _sp_view(a):
        return a.reshape(-1, SCH)

    zeros_deg = jnp.zeros((DPT,), f32)
    zeros_row = jnp.zeros((WB, F), f32)

    deg_s, deg_d, deg_c = _make_deg((es_pad, ed_pad, ec_pad))(
        _deg_view(col_s, es_pad), _deg_view(col_d, ed_pad),
        _deg_view(col_c, ec_pad), ew_s, ew_d, ew_c, zeros_deg)
    degT_s = jnp.swapaxes(deg_s, 0, 1)[:N]
    degT_d = jnp.swapaxes(deg_d, 0, 1)[:N]
    degT_c = jnp.swapaxes(deg_c, 0, 1)[:N]

    grid = (N // RB,)
    y_s, y_d, y_c = pl.pallas_call(
        _tc_pre_body,
        grid=grid,
        in_specs=[_deg_spec(), _deg_spec(), _deg_spec(),
                  _row_spec(F), _row_spec(F)],
        out_specs=[_row_spec(F)] * 3,
        out_shape=[jax.ShapeDtypeStruct((N, F), f32)] * 3,
    )(degT_s, degT_d, degT_c, x_RNA, x_ADT)

    spmm_sim = _make_spmm(es_pad, True)
    spmm_dist = _make_spmm(ed_pad, True)
    spmm_com = _make_spmm(ec_pad, False)

    p_s = spmm_sim(y_s, _sp_view(row_s), _sp_view(col_s),
                   _sp_view(ew_s), zeros_row)
    p_d = spmm_dist(y_d, _sp_view(row_d), _sp_view(col_d),
                    _sp_view(ew_d), zeros_row)
    p_c = spmm_com(y_c, _sp_view(row_c), _sp_view(col_c),
                   _sp_view(ew_c), zeros_row)

    def mid(p, degT, w1, b1, w2):
        return pl.pallas_call(
            _tc_mid_body,
            grid=grid,
            in_specs=[_part_spec(), _row_spec(F), _deg_spec(),
                      _full_spec((F, HID)), _full_spec((1, HID)),
                      _full_spec((HID, F))],
            out_specs=[_row_spec(F), _row_spec(F)],
            out_shape=[jax.ShapeDtypeStruct((N, F), f32)] * 2,
        )(p, x_RNA, degT, w1, b1.reshape(1, HID), w2)

    ys, ysc = mid(p_s, degT_s, W_r1, b_r1, W_sim)
    yd, ydc = mid(p_d, degT_d, W_r2, b_r2, W_dist)

    q_s = spmm_sim(ysc, _sp_view(row_s), _sp_view(col_s),
                   _sp_view(ew_s), zeros_row)
    q_d = spmm_dist(ydc, _sp_view(row_d), _sp_view(col_d),
                    _sp_view(ew_d), zeros_row)

    outs = pl.pallas_call(
        _tc_final_body,
        grid=grid,
        in_specs=[_part_spec(), _part_spec(), _part_spec(),
                  _row_spec(F), _row_spec(F), _row_spec(F),
                  _deg_spec(), _deg_spec(), _deg_spec(),
                  _full_spec((F, F)), _full_spec((1, F)),
                  _full_spec((1, F)), _full_spec((1, F)),
                  _full_spec((F, F)), _full_spec((F, F)), _full_spec((1, F)),
                  _full_spec((F, F)), _full_spec((F, F)), _full_spec((1, F))],
        out_specs=[_row_spec(F)] * 5,
        out_shape=[jax.ShapeDtypeStruct((N, F), f32)] * 5,
    )(q_s, q_d, p_c, ys, yd, x_ADT, degT_s, degT_d, degT_c,
      W_p3, b_p3.reshape(1, F), b_sim.reshape(1, F), b_dist.reshape(1, F),
      W_f1[:F], W_f1[F:], b_f1.reshape(1, F),
      W_f2[:F], W_f2[F:], b_f2.reshape(1, F))
    x_sim, x_dist, fused, fused_pro, pro = outs
    return (x_sim, x_dist, fused, fused_pro, pro)
